# baseline (speedup 1.0000x reference)
import functools

import jax
import jax.numpy as jnp
from jax import lax
from jax.experimental import pallas as pl
from jax.experimental.pallas import tpu as pltpu
from jax.experimental.pallas import tpu_sc as plsc

B_MAX = 10.0
NUM_REL = 1_000_000

NC, NS, L = 2, 16, 16
NW = NC * NS

ROWS, COLS = 16384, 200
NSPLIT = 2
ROWS_H = ROWS // NSPLIT
TOTAL = ROWS_H * COLS
PER_W = TOTAL // NW
CHUNK = 12_800
NCHUNK = PER_W // CHUNK
NSTREAM = 8
SUB = CHUNK // NSTREAM
UNROLL = 4

TBL_WIN = 64_000
LAST_WIN = NUM_REL - TBL_WIN


def _body(ids_hbm, w_hbm, out_hbm, tbl_s, idx0, idx1, val0, val1,
          sem_i, sem_g, sem_o):
    cid = lax.axis_index("c")
    sid = lax.axis_index("s")
    idxb = (idx0, idx1)
    valb = (val0, val1)

    wid = sid * NC + cid
    base = wid * PER_W

    def ids_start(c):
        pltpu.async_copy(ids_hbm.at[pl.ds(base + c * CHUNK, CHUNK)],
                         idxb[c % 2], sem_i)

    def wait_ids(c):
        pltpu.make_async_copy(ids_hbm.at[pl.ds(0, CHUNK)], idxb[c % 2],
                              sem_i).wait()

    def wait_out(c):
        pltpu.make_async_copy(valb[c % 2], out_hbm.at[pl.ds(0, CHUNK)],
                              sem_o).wait()

    ids_start(0)
    ids_start(1)

    t0 = jnp.minimum(sid * TBL_WIN, LAST_WIN)
    pltpu.sync_copy(w_hbm.at[pl.ds(t0, TBL_WIN)], tbl_s.at[pl.ds(t0, TBL_WIN)])
    plsc.subcore_barrier()

    def gathers(c):
        ib, vb = idxb[c % 2], valb[c % 2]
        for k in range(NSTREAM):
            pltpu.async_copy(tbl_s.at[ib.at[pl.ds(k * SUB, SUB)]],
                             vb.at[pl.ds(k * SUB, SUB)], sem_g)

    def drain_gathers(c):
        pltpu.make_async_copy(ids_hbm.at[pl.ds(0, CHUNK)], valb[c % 2],
                              sem_g).wait()

    def sigmoid_and_out(c):
        buf = valb[c % 2]

        def blk(i, carry):
            for u in range(UNROLL):
                o = (i * UNROLL + u) * L
                x = buf[pl.ds(o, L)]
                buf[pl.ds(o, L)] = B_MAX / (1.0 + jnp.exp(-x))
            return carry

        lax.fori_loop(0, CHUNK // (L * UNROLL), blk, 0)
        pltpu.async_copy(buf, out_hbm.at[pl.ds(base + c * CHUNK, CHUNK)],
                         sem_o)

    for c in range(NCHUNK + 1):
        if c < NCHUNK:
            if c >= 2:
                wait_out(c - 2)
            wait_ids(c)
            gathers(c)
            if 1 <= c < NCHUNK - 1:
                ids_start(c + 1)
        if c >= 1:
            sigmoid_and_out(c - 1)
        if c < NCHUNK:
            drain_gathers(c)

    wait_out(NCHUNK - 2)
    wait_out(NCHUNK - 1)


_mesh = plsc.VectorSubcoreMesh(core_axis_name="c", subcore_axis_name="s")

_sc_call = functools.partial(
    pl.kernel,
    out_type=jax.ShapeDtypeStruct((TOTAL,), jnp.float32),
    mesh=_mesh,
    scratch_types=[
        pltpu.VMEM_SHARED((NUM_REL,), jnp.float32),
        pltpu.VMEM((CHUNK,), jnp.int32),
        pltpu.VMEM((CHUNK,), jnp.int32),
        pltpu.VMEM((CHUNK,), jnp.float32),
        pltpu.VMEM((CHUNK,), jnp.float32),
        pltpu.SemaphoreType.DMA,
        pltpu.SemaphoreType.DMA,
        pltpu.SemaphoreType.DMA,
    ],
    compiler_params=pltpu.CompilerParams(use_tc_tiling_on_sc=False),
)(_body)


def kernel(r_ids, w):
    w_flat = w.reshape(-1)
    ids = r_ids.astype(jnp.int32)
    outs = []
    for h in range(NSPLIT):
        ids_h = ids[h * ROWS_H:(h + 1) * ROWS_H].reshape(-1)
        outs.append(_sc_call(ids_h, w_flat).reshape(ROWS_H, COLS))
    return jnp.concatenate(outs, axis=0)


# speedup vs baseline: 1.1302x; 1.1302x over previous
import functools

import jax
import jax.numpy as jnp
from jax import lax
from jax.experimental import pallas as pl
from jax.experimental.pallas import tpu as pltpu
from jax.experimental.pallas import tpu_sc as plsc

B_MAX = 10.0
NUM_REL = 1_000_000

NC, NS, L = 2, 16, 16
NW = NC * NS

ROWS, COLS = 16384, 200
TOTAL = ROWS * COLS
PER_W = TOTAL // NW
CHUNK = 12_800
NCHUNK = PER_W // CHUNK
NSTREAM = 8
SUB = CHUNK // NSTREAM
UNROLL = 4

TBL_WIN = 64_000
LAST_WIN = NUM_REL - TBL_WIN


def _body(ids_hbm, w_hbm, out_hbm, tbl_s, idx0, idx1, val0, val1,
          sem_i, sem_g, sem_o):
    cid = lax.axis_index("c")
    sid = lax.axis_index("s")
    idxb = (idx0, idx1)
    valb = (val0, val1)

    wid = sid * NC + cid
    base = wid * PER_W

    def ids_start(c):
        pltpu.async_copy(ids_hbm.at[pl.ds(base + c * CHUNK, CHUNK)],
                         idxb[c % 2], sem_i)

    def wait_ids(c):
        pltpu.make_async_copy(ids_hbm.at[pl.ds(0, CHUNK)], idxb[c % 2],
                              sem_i).wait()

    def wait_out(c):
        pltpu.make_async_copy(valb[c % 2], out_hbm.at[pl.ds(0, CHUNK)],
                              sem_o).wait()

    ids_start(0)
    ids_start(1)

    t0 = jnp.minimum(sid * TBL_WIN, LAST_WIN)
    pltpu.sync_copy(w_hbm.at[pl.ds(t0, TBL_WIN)], tbl_s.at[pl.ds(t0, TBL_WIN)])
    plsc.subcore_barrier()

    def gathers(c):
        ib, vb = idxb[c % 2], valb[c % 2]
        for k in range(NSTREAM):
            pltpu.async_copy(tbl_s.at[ib.at[pl.ds(k * SUB, SUB)]],
                             vb.at[pl.ds(k * SUB, SUB)], sem_g)

    def drain_gathers(c):
        pltpu.make_async_copy(ids_hbm.at[pl.ds(0, CHUNK)], valb[c % 2],
                              sem_g).wait()

    def sigmoid_and_out(c):
        buf = valb[c % 2]

        def blk(i, carry):
            for u in range(UNROLL):
                o = (i * UNROLL + u) * L
                x = buf[pl.ds(o, L)]
                buf[pl.ds(o, L)] = B_MAX / (1.0 + jnp.exp(-x))
            return carry

        lax.fori_loop(0, CHUNK // (L * UNROLL), blk, 0)
        pltpu.async_copy(buf, out_hbm.at[pl.ds(base + c * CHUNK, CHUNK)],
                         sem_o)

    for c in range(NCHUNK + 1):
        if c < NCHUNK:
            if c >= 2:
                wait_out(c - 2)
            wait_ids(c)
            gathers(c)
            if 1 <= c < NCHUNK - 1:
                ids_start(c + 1)
        if c >= 1:
            sigmoid_and_out(c - 1)
        if c < NCHUNK:
            drain_gathers(c)

    wait_out(NCHUNK - 2)
    wait_out(NCHUNK - 1)


_mesh = plsc.VectorSubcoreMesh(core_axis_name="c", subcore_axis_name="s")

_sc_call = functools.partial(
    pl.kernel,
    out_type=jax.ShapeDtypeStruct((TOTAL,), jnp.float32),
    mesh=_mesh,
    scratch_types=[
        pltpu.VMEM_SHARED((NUM_REL,), jnp.float32),
        pltpu.VMEM((CHUNK,), jnp.int32),
        pltpu.VMEM((CHUNK,), jnp.int32),
        pltpu.VMEM((CHUNK,), jnp.float32),
        pltpu.VMEM((CHUNK,), jnp.float32),
        pltpu.SemaphoreType.DMA,
        pltpu.SemaphoreType.DMA,
        pltpu.SemaphoreType.DMA,
    ],
    compiler_params=pltpu.CompilerParams(use_tc_tiling_on_sc=False),
)(_body)


def kernel(r_ids, w):
    out = _sc_call(r_ids.reshape(-1), w.reshape(-1))
    return out.reshape(ROWS, COLS)


# speedup vs baseline: 1.1628x; 1.0289x over previous
import functools

import jax
import jax.numpy as jnp
from jax import lax
from jax.experimental import pallas as pl
from jax.experimental.pallas import tpu as pltpu
from jax.experimental.pallas import tpu_sc as plsc

B_MAX = 10.0
NUM_REL = 1_000_000

NC, NS, L = 2, 16, 16
NW = NC * NS

ROWS, COLS = 16384, 200
TOTAL = ROWS * COLS
PER_W = TOTAL // NW
CHUNK = 12_800
NCHUNK = PER_W // CHUNK
NSTREAM = 16
SUB = CHUNK // NSTREAM
UNROLL = 8

TBL_WIN = 64_000
LAST_WIN = NUM_REL - TBL_WIN


def _body(ids_hbm, w_hbm, out_hbm, tbl_s, idx0, idx1, val0, val1,
          sem_i, sem_g, sem_o):
    cid = lax.axis_index("c")
    sid = lax.axis_index("s")
    idxb = (idx0, idx1)
    valb = (val0, val1)

    wid = sid * NC + cid
    base = wid * PER_W

    def ids_start(c):
        pltpu.async_copy(ids_hbm.at[pl.ds(base + c * CHUNK, CHUNK)],
                         idxb[c % 2], sem_i)

    def wait_ids(c):
        pltpu.make_async_copy(ids_hbm.at[pl.ds(0, CHUNK)], idxb[c % 2],
                              sem_i).wait()

    def wait_out(c):
        pltpu.make_async_copy(valb[c % 2], out_hbm.at[pl.ds(0, CHUNK)],
                              sem_o).wait()

    ids_start(0)
    ids_start(1)

    t0 = jnp.minimum(sid * TBL_WIN, LAST_WIN)
    pltpu.sync_copy(w_hbm.at[pl.ds(t0, TBL_WIN)], tbl_s.at[pl.ds(t0, TBL_WIN)])
    plsc.subcore_barrier()

    def gathers(c):
        ib, vb = idxb[c % 2], valb[c % 2]
        for k in range(NSTREAM):
            pltpu.async_copy(tbl_s.at[ib.at[pl.ds(k * SUB, SUB)]],
                             vb.at[pl.ds(k * SUB, SUB)], sem_g)

    def drain_gathers(c):
        pltpu.make_async_copy(ids_hbm.at[pl.ds(0, CHUNK)], valb[c % 2],
                              sem_g).wait()

    def sigmoid_and_out(c):
        buf = valb[c % 2]

        def blk(i, carry):
            for u in range(UNROLL):
                o = (i * UNROLL + u) * L
                x = buf[pl.ds(o, L)]
                buf[pl.ds(o, L)] = B_MAX / (1.0 + jnp.exp(-x))
            return carry

        lax.fori_loop(0, CHUNK // (L * UNROLL), blk, 0)
        pltpu.async_copy(buf, out_hbm.at[pl.ds(base + c * CHUNK, CHUNK)],
                         sem_o)

    for c in range(NCHUNK + 1):
        if c < NCHUNK:
            if c >= 2:
                wait_out(c - 2)
            wait_ids(c)
            gathers(c)
            if 1 <= c < NCHUNK - 1:
                ids_start(c + 1)
        if c >= 1:
            sigmoid_and_out(c - 1)
        if c < NCHUNK:
            drain_gathers(c)

    wait_out(NCHUNK - 2)
    wait_out(NCHUNK - 1)


_mesh = plsc.VectorSubcoreMesh(core_axis_name="c", subcore_axis_name="s")

_sc_call = functools.partial(
    pl.kernel,
    out_type=jax.ShapeDtypeStruct((TOTAL,), jnp.float32),
    mesh=_mesh,
    scratch_types=[
        pltpu.VMEM_SHARED((NUM_REL,), jnp.float32),
        pltpu.VMEM((CHUNK,), jnp.int32),
        pltpu.VMEM((CHUNK,), jnp.int32),
        pltpu.VMEM((CHUNK,), jnp.float32),
        pltpu.VMEM((CHUNK,), jnp.float32),
        pltpu.SemaphoreType.DMA,
        pltpu.SemaphoreType.DMA,
        pltpu.SemaphoreType.DMA,
    ],
    compiler_params=pltpu.CompilerParams(use_tc_tiling_on_sc=False),
)(_body)


def kernel(r_ids, w):
    out = _sc_call(r_ids.reshape(-1), w.reshape(-1))
    return out.reshape(ROWS, COLS)
